# initial kernel scaffold (unmeasured)
import jax
import jax.numpy as jnp
from jax import lax
from jax.experimental import pallas as pl
from jax.experimental.pallas import tpu as pltpu


def kernel(x, assign, W1, W2):
    m, d = x.shape
    ne, _, f = W1.shape
    assign2 = assign.reshape(m, 1)

    def body(x_ref, a_ref, w1_ref, w2_ref, out_ref,
             xg, ag, acc, orecv, send_sems, recv_sems):
        my_x = lax.axis_index("x")
        my_y = lax.axis_index("y")
        peer = (my_x, 1 - my_y)

        barrier = pltpu.get_barrier_semaphore()
        pl.semaphore_signal(barrier, inc=1, device_id=peer,
                            device_id_type=pl.DeviceIdType.MESH)
        pl.semaphore_wait(barrier, 1)

        xg[0, :, :] = x_ref[:, :]
        ag[0, :, :] = a_ref[:, :]

        rdma_x = pltpu.make_async_remote_copy(
            src_ref=x_ref, dst_ref=xg.at[1],
            send_sem=send_sems.at[0], recv_sem=recv_sems.at[0],
            device_id=peer, device_id_type=pl.DeviceIdType.MESH)
        rdma_x.start()
        rdma_a = pltpu.make_async_remote_copy(
            src_ref=a_ref, dst_ref=ag.at[1],
            send_sem=send_sems.at[1], recv_sem=recv_sems.at[1],
            device_id=peer, device_id_type=pl.DeviceIdType.MESH)
        rdma_a.start()
        rdma_x.wait()
        rdma_a.wait()

        for slot in range(2):
            xs = xg[slot, :, :]
            aslot = ag[slot, :, :]
            total = None
            for e in range(ne):
                e_glob = my_y * ne + e
                h = jnp.maximum(
                    jnp.dot(xs, w1_ref[e, :, :],
                            preferred_element_type=jnp.float32), 0.0)
                o = jnp.dot(h, w2_ref[e, :, :],
                            preferred_element_type=jnp.float32)
                contrib = jnp.where(aslot == e_glob, o, 0.0)
                total = contrib if total is None else total + contrib
            acc[slot, :, :] = total

        rdma_o = pltpu.make_async_remote_copy(
            src_ref=acc.at[1], dst_ref=orecv,
            send_sem=send_sems.at[2], recv_sem=recv_sems.at[2],
            device_id=peer, device_id_type=pl.DeviceIdType.MESH)
        rdma_o.start()
        rdma_o.wait()

        out_ref[:, :] = acc[0, :, :] + orecv[:, :]

    return pl.pallas_call(
        body,
        out_shape=jax.ShapeDtypeStruct((m, d), jnp.float32),
        in_specs=[
            pl.BlockSpec(memory_space=pltpu.VMEM),
            pl.BlockSpec(memory_space=pltpu.VMEM),
            pl.BlockSpec(memory_space=pltpu.VMEM),
            pl.BlockSpec(memory_space=pltpu.VMEM),
        ],
        out_specs=pl.BlockSpec(memory_space=pltpu.VMEM),
        scratch_shapes=[
            pltpu.VMEM((2, m, d), jnp.float32),
            pltpu.VMEM((2, m, 1), jnp.int32),
            pltpu.VMEM((2, m, d), jnp.float32),
            pltpu.VMEM((m, d), jnp.float32),
            pltpu.SemaphoreType.DMA((3,)),
            pltpu.SemaphoreType.DMA((3,)),
        ],
        compiler_params=pltpu.CompilerParams(collective_id=0),
    )(x, assign2, W1, W2)


# baseline (device time: 161131 ns/iter reference)
import jax
import jax.numpy as jnp
from jax import lax
from jax.experimental import pallas as pl
from jax.experimental.pallas import tpu as pltpu

F_BLK = 512


def kernel(x, assign, W1, W2):
    m, d = x.shape
    ne, _, f = W1.shape
    nfc = f // F_BLK
    assign2 = assign.reshape(m, 1)

    def body(x_ref, a_ref, w1_ref, w2_ref, out_ref,
             xg, ag, acc, orecv, send_sems, recv_sems):
        e = pl.program_id(0)
        fc = pl.program_id(1)
        first = jnp.logical_and(e == 0, fc == 0)
        last = jnp.logical_and(e == ne - 1, fc == nfc - 1)

        my_x = lax.axis_index("x")
        my_y = lax.axis_index("y")
        peer = (my_x, 1 - my_y)

        @pl.when(first)
        def _exchange_tokens():
            barrier = pltpu.get_barrier_semaphore()
            pl.semaphore_signal(barrier, inc=1, device_id=peer,
                                device_id_type=pl.DeviceIdType.MESH)
            pl.semaphore_wait(barrier, 1)

            xg[0, :, :] = x_ref[:, :]
            ag[0, :, :] = a_ref[:, :]

            rdma_x = pltpu.make_async_remote_copy(
                src_ref=x_ref, dst_ref=xg.at[1],
                send_sem=send_sems.at[0], recv_sem=recv_sems.at[0],
                device_id=peer, device_id_type=pl.DeviceIdType.MESH)
            rdma_x.start()
            rdma_a = pltpu.make_async_remote_copy(
                src_ref=a_ref, dst_ref=ag.at[1],
                send_sem=send_sems.at[1], recv_sem=recv_sems.at[1],
                device_id=peer, device_id_type=pl.DeviceIdType.MESH)
            rdma_a.start()
            rdma_x.wait()
            rdma_a.wait()

        e_glob = my_y * ne + e
        w1 = w1_ref[0, :, :]
        w2 = w2_ref[0, :, :]
        for slot in range(2):
            h = jnp.maximum(
                jnp.dot(xg[slot, :, :], w1,
                        preferred_element_type=jnp.float32), 0.0)
            o = jnp.dot(h, w2, preferred_element_type=jnp.float32)
            contrib = jnp.where(ag[slot, :, :] == e_glob, o, 0.0)

            @pl.when(first)
            def _init():
                acc[slot, :, :] = contrib

            @pl.when(jnp.logical_not(first))
            def _accum():
                acc[slot, :, :] = acc[slot, :, :] + contrib

        @pl.when(last)
        def _exchange_out():
            rdma_o = pltpu.make_async_remote_copy(
                src_ref=acc.at[1], dst_ref=orecv,
                send_sem=send_sems.at[2], recv_sem=recv_sems.at[2],
                device_id=peer, device_id_type=pl.DeviceIdType.MESH)
            rdma_o.start()
            rdma_o.wait()
            out_ref[:, :] = acc[0, :, :] + orecv[:, :]

    return pl.pallas_call(
        body,
        grid=(ne, nfc),
        out_shape=jax.ShapeDtypeStruct((m, d), jnp.float32),
        in_specs=[
            pl.BlockSpec((m, d), lambda e, fc: (0, 0),
                         memory_space=pltpu.VMEM),
            pl.BlockSpec((m, 1), lambda e, fc: (0, 0),
                         memory_space=pltpu.VMEM),
            pl.BlockSpec((1, d, F_BLK), lambda e, fc: (e, 0, fc),
                         memory_space=pltpu.VMEM),
            pl.BlockSpec((1, F_BLK, d), lambda e, fc: (e, fc, 0),
                         memory_space=pltpu.VMEM),
        ],
        out_specs=pl.BlockSpec((m, d), lambda e, fc: (0, 0),
                               memory_space=pltpu.VMEM),
        scratch_shapes=[
            pltpu.VMEM((2, m, d), jnp.float32),
            pltpu.VMEM((2, m, 1), jnp.int32),
            pltpu.VMEM((2, m, d), jnp.float32),
            pltpu.VMEM((m, d), jnp.float32),
            pltpu.SemaphoreType.DMA((3,)),
            pltpu.SemaphoreType.DMA((3,)),
        ],
        compiler_params=pltpu.CompilerParams(
            collective_id=0,
            dimension_semantics=("arbitrary", "arbitrary"),
            vmem_limit_bytes=60 * 1024 * 1024,
        ),
    )(x, assign2, W1, W2)


# device time: 92971 ns/iter; 1.7331x vs baseline; 1.7331x over previous
import jax
import jax.numpy as jnp
from jax import lax
from jax.experimental import pallas as pl
from jax.experimental.pallas import tpu as pltpu

F_BLK = 512
RB = 512
N_RB = 2


def kernel(x, assign, W1, W2):
    m, d = x.shape
    ne, _, f = W1.shape
    nfc = f // F_BLK
    assign2 = assign.reshape(m, 1)

    def body(x_ref, a_ref, w1_ref, w2_ref, out_ref,
             xbf, xg, ag, acc, obf, orecv, send_sems, recv_sems):
        slot = pl.program_id(0)
        rb = pl.program_id(1)
        e = pl.program_id(2)
        fc = pl.program_id(3)
        first = (slot == 0) & (rb == 0) & (e == 0) & (fc == 0)
        last_ef = (e == ne - 1) & (fc == nfc - 1)
        final = (slot == 1) & (rb == N_RB - 1) & last_ef

        my_x = lax.axis_index("x")
        my_y = lax.axis_index("y")
        peer = (my_x, 1 - my_y)

        def token_rdma(rbv):
            rows = pl.ds(rbv * RB, RB)
            return pltpu.make_async_remote_copy(
                src_ref=xbf.at[rows, :], dst_ref=xg.at[rows, :],
                send_sem=send_sems.at[rbv], recv_sem=recv_sems.at[rbv],
                device_id=peer, device_id_type=pl.DeviceIdType.MESH)

        def assign_rdma():
            return pltpu.make_async_remote_copy(
                src_ref=a_ref, dst_ref=ag,
                send_sem=send_sems.at[N_RB], recv_sem=recv_sems.at[N_RB],
                device_id=peer, device_id_type=pl.DeviceIdType.MESH)

        def out_rdma(rbv):
            rows = pl.ds(rbv * RB, RB)
            return pltpu.make_async_remote_copy(
                src_ref=obf.at[rows, :], dst_ref=orecv.at[rows, :],
                send_sem=send_sems.at[N_RB + 1 + rbv],
                recv_sem=recv_sems.at[N_RB + 1 + rbv],
                device_id=peer, device_id_type=pl.DeviceIdType.MESH)

        @pl.when(first)
        def _start_token_exchange():
            barrier = pltpu.get_barrier_semaphore()
            pl.semaphore_signal(barrier, inc=1, device_id=peer,
                                device_id_type=pl.DeviceIdType.MESH)
            pl.semaphore_wait(barrier, 1)
            xbf[:, :] = x_ref[:, :].astype(jnp.bfloat16)
            for rbv in range(N_RB):
                token_rdma(rbv).start()
            assign_rdma().start()

        e_glob = my_y * ne + e
        w1b = w1_ref[0, :, :].astype(jnp.bfloat16)
        w2b = w2_ref[0, :, :].astype(jnp.bfloat16)
        rows = pl.ds(rb * RB, RB)

        def moe_chunk(sl, xs, mask):
            h = jnp.maximum(
                jnp.dot(xs, w1b, preferred_element_type=jnp.float32), 0.0)
            o = jnp.dot(h.astype(jnp.bfloat16), w2b,
                        preferred_element_type=jnp.float32)
            contrib = jnp.where(mask, o, 0.0)

            @pl.when((e == 0) & (fc == 0))
            def _init():
                acc[sl, rows, :] = contrib

            @pl.when(~((e == 0) & (fc == 0)))
            def _accum():
                acc[sl, rows, :] = acc[sl, rows, :] + contrib

        @pl.when(slot == 0)
        def _local_block():
            moe_chunk(0, xbf[rows, :], a_ref[rows, :] == e_glob)

        @pl.when(slot == 1)
        def _remote_block():
            @pl.when((rb == 0) & (e == 0) & (fc == 0))
            def _():
                token_rdma(0).wait()
                assign_rdma().wait()

            @pl.when((rb == 1) & (e == 0) & (fc == 0))
            def _():
                token_rdma(1).wait()

            moe_chunk(1, xg[rows, :], ag[rows, :] == e_glob)

            @pl.when(last_ef)
            def _send_partial():
                obf[rows, :] = acc[1, rows, :].astype(jnp.bfloat16)
                for rbv in range(N_RB):
                    @pl.when(rb == rbv)
                    def _(rbv=rbv):
                        out_rdma(rbv).start()

        @pl.when(final)
        def _finish():
            for rbv in range(N_RB):
                out_rdma(rbv).wait()
            out_ref[:, :] = acc[0, :, :] + orecv[:, :].astype(jnp.float32)

    return pl.pallas_call(
        body,
        grid=(2, N_RB, ne, nfc),
        out_shape=jax.ShapeDtypeStruct((m, d), jnp.float32),
        in_specs=[
            pl.BlockSpec((m, d), lambda s, r, e, fc: (0, 0),
                         memory_space=pltpu.VMEM),
            pl.BlockSpec((m, 1), lambda s, r, e, fc: (0, 0),
                         memory_space=pltpu.VMEM),
            pl.BlockSpec((1, d, F_BLK), lambda s, r, e, fc: (e, 0, fc),
                         memory_space=pltpu.VMEM),
            pl.BlockSpec((1, F_BLK, d), lambda s, r, e, fc: (e, fc, 0),
                         memory_space=pltpu.VMEM),
        ],
        out_specs=pl.BlockSpec((m, d), lambda s, r, e, fc: (0, 0),
                               memory_space=pltpu.VMEM),
        scratch_shapes=[
            pltpu.VMEM((m, d), jnp.bfloat16),
            pltpu.VMEM((m, d), jnp.bfloat16),
            pltpu.VMEM((m, 1), jnp.int32),
            pltpu.VMEM((2, m, d), jnp.float32),
            pltpu.VMEM((m, d), jnp.bfloat16),
            pltpu.VMEM((m, d), jnp.bfloat16),
            pltpu.SemaphoreType.DMA((N_RB + 1 + N_RB,)),
            pltpu.SemaphoreType.DMA((N_RB + 1 + N_RB,)),
        ],
        compiler_params=pltpu.CompilerParams(
            collective_id=0,
            dimension_semantics=("arbitrary",) * 4,
            vmem_limit_bytes=60 * 1024 * 1024,
        ),
    )(x, assign2, W1, W2)


# device time: 88371 ns/iter; 1.8233x vs baseline; 1.0521x over previous
import jax
import jax.numpy as jnp
from jax import lax
from jax.experimental import pallas as pl
from jax.experimental.pallas import tpu as pltpu

F_BLK = 512
RB = 512
N_RB = 2


def kernel(x, assign, W1, W2):
    m, d = x.shape
    ne, _, f = W1.shape
    nfc = f // F_BLK
    assign2 = assign.reshape(m, 1)

    def body(x_ref, a_ref, w1_ref, w2_ref, out_ref,
             xbf, xg, ag, acc, obf, orecv, w1bf, w2bf,
             send_sems, recv_sems):
        slot = pl.program_id(0)
        rb = pl.program_id(1)
        e = pl.program_id(2)
        fc = pl.program_id(3)
        first = (slot == 0) & (rb == 0) & (e == 0) & (fc == 0)
        last_ef = (e == ne - 1) & (fc == nfc - 1)
        final = (slot == 1) & (rb == N_RB - 1) & last_ef

        my_x = lax.axis_index("x")
        my_y = lax.axis_index("y")
        peer = (my_x, 1 - my_y)

        def token_rdma(rbv):
            rows = pl.ds(rbv * RB, RB)
            return pltpu.make_async_remote_copy(
                src_ref=xbf.at[rows, :], dst_ref=xg.at[rows, :],
                send_sem=send_sems.at[rbv], recv_sem=recv_sems.at[rbv],
                device_id=peer, device_id_type=pl.DeviceIdType.MESH)

        def assign_rdma():
            return pltpu.make_async_remote_copy(
                src_ref=a_ref, dst_ref=ag,
                send_sem=send_sems.at[N_RB], recv_sem=recv_sems.at[N_RB],
                device_id=peer, device_id_type=pl.DeviceIdType.MESH)

        def out_rdma(rbv):
            rows = pl.ds(rbv * RB, RB)
            return pltpu.make_async_remote_copy(
                src_ref=obf.at[rows, :], dst_ref=orecv.at[rows, :],
                send_sem=send_sems.at[N_RB + 1 + rbv],
                recv_sem=recv_sems.at[N_RB + 1 + rbv],
                device_id=peer, device_id_type=pl.DeviceIdType.MESH)

        @pl.when(first)
        def _start_token_exchange():
            barrier = pltpu.get_barrier_semaphore()
            pl.semaphore_signal(barrier, inc=1, device_id=peer,
                                device_id_type=pl.DeviceIdType.MESH)
            pl.semaphore_wait(barrier, 1)
            xbf[:, :] = x_ref[:, :].astype(jnp.bfloat16)
            for rbv in range(N_RB):
                token_rdma(rbv).start()
            assign_rdma().start()

        e_glob = my_y * ne + e
        fcols = pl.ds(fc * F_BLK, F_BLK)

        @pl.when((slot == 0) & (rb == 0))
        def _cache_weights():
            w1bf[e, :, fcols] = w1_ref[0, :, :].astype(jnp.bfloat16)
            w2bf[e, fcols, :] = w2_ref[0, :, :].astype(jnp.bfloat16)

        w1b = w1bf[e, :, fcols]
        w2b = w2bf[e, fcols, :]
        rows = pl.ds(rb * RB, RB)

        def moe_chunk(sl, xs, mask):
            h = jnp.maximum(
                jnp.dot(xs, w1b, preferred_element_type=jnp.float32), 0.0)
            o = jnp.dot(h.astype(jnp.bfloat16), w2b,
                        preferred_element_type=jnp.float32)
            contrib = jnp.where(mask, o, 0.0)

            @pl.when((e == 0) & (fc == 0))
            def _init():
                acc[sl, rows, :] = contrib

            @pl.when(~((e == 0) & (fc == 0)))
            def _accum():
                acc[sl, rows, :] = acc[sl, rows, :] + contrib

        @pl.when(slot == 0)
        def _local_block():
            moe_chunk(0, xbf[rows, :], a_ref[rows, :] == e_glob)

        @pl.when(slot == 1)
        def _remote_block():
            @pl.when((rb == 0) & (e == 0) & (fc == 0))
            def _():
                token_rdma(0).wait()
                assign_rdma().wait()

            @pl.when((rb == 1) & (e == 0) & (fc == 0))
            def _():
                token_rdma(1).wait()

            moe_chunk(1, xg[rows, :], ag[rows, :] == e_glob)

            @pl.when(last_ef)
            def _send_partial():
                obf[rows, :] = acc[1, rows, :].astype(jnp.bfloat16)
                for rbv in range(N_RB):
                    @pl.when(rb == rbv)
                    def _(rbv=rbv):
                        out_rdma(rbv).start()

        @pl.when(final)
        def _finish():
            for rbv in range(N_RB):
                out_rdma(rbv).wait()
            out_ref[:, :] = acc[0, :, :] + orecv[:, :].astype(jnp.float32)

    return pl.pallas_call(
        body,
        grid=(2, N_RB, ne, nfc),
        out_shape=jax.ShapeDtypeStruct((m, d), jnp.float32),
        in_specs=[
            pl.BlockSpec((m, d), lambda s, r, e, fc: (0, 0),
                         memory_space=pltpu.VMEM),
            pl.BlockSpec((m, 1), lambda s, r, e, fc: (0, 0),
                         memory_space=pltpu.VMEM),
            pl.BlockSpec(
                (1, d, F_BLK),
                lambda s, r, e, fc: (
                    jnp.where((s == 0) & (r == 0), e, 1),
                    0,
                    jnp.where((s == 0) & (r == 0), fc, 3),
                ),
                memory_space=pltpu.VMEM),
            pl.BlockSpec(
                (1, F_BLK, d),
                lambda s, r, e, fc: (
                    jnp.where((s == 0) & (r == 0), e, 1),
                    jnp.where((s == 0) & (r == 0), fc, 3),
                    0,
                ),
                memory_space=pltpu.VMEM),
        ],
        out_specs=pl.BlockSpec((m, d), lambda s, r, e, fc: (0, 0),
                               memory_space=pltpu.VMEM),
        scratch_shapes=[
            pltpu.VMEM((m, d), jnp.bfloat16),
            pltpu.VMEM((m, d), jnp.bfloat16),
            pltpu.VMEM((m, 1), jnp.int32),
            pltpu.VMEM((2, m, d), jnp.float32),
            pltpu.VMEM((m, d), jnp.bfloat16),
            pltpu.VMEM((m, d), jnp.bfloat16),
            pltpu.VMEM((ne, d, f), jnp.bfloat16),
            pltpu.VMEM((ne, f, d), jnp.bfloat16),
            pltpu.SemaphoreType.DMA((N_RB + 1 + N_RB,)),
            pltpu.SemaphoreType.DMA((N_RB + 1 + N_RB,)),
        ],
        compiler_params=pltpu.CompilerParams(
            collective_id=0,
            dimension_semantics=("arbitrary",) * 4,
            vmem_limit_bytes=60 * 1024 * 1024,
        ),
    )(x, assign2, W1, W2)


# device time: 85863 ns/iter; 1.8766x vs baseline; 1.0292x over previous
import jax
import jax.numpy as jnp
from jax import lax
from jax.experimental import pallas as pl
from jax.experimental.pallas import tpu as pltpu

F_BLK = 512
RB = 512
N_RB = 2


def kernel(x, assign, W1, W2):
    m, d = x.shape
    ne, _, f = W1.shape
    nfc = f // F_BLK
    assign2 = assign.reshape(m, 1)

    def body(x_ref, a_ref, w1_ref, w2_ref, out_ref,
             xbf, xg, ag, acc, obf, orecv, w1bf, w2bf, hbuf,
             send_sems, recv_sems):
        slot = pl.program_id(0)
        rb = pl.program_id(1)
        e = pl.program_id(2)
        fc = pl.program_id(3)
        first = (slot == 0) & (rb == 0) & (e == 0) & (fc == 0)
        last_ef = (e == ne - 1) & (fc == nfc - 1)
        final = (slot == 1) & (rb == N_RB - 1) & last_ef

        my_x = lax.axis_index("x")
        my_y = lax.axis_index("y")
        peer = (my_x, 1 - my_y)

        def token_rdma(rbv):
            rows = pl.ds(rbv * RB, RB)
            return pltpu.make_async_remote_copy(
                src_ref=xbf.at[rows, :], dst_ref=xg.at[rows, :],
                send_sem=send_sems.at[rbv], recv_sem=recv_sems.at[rbv],
                device_id=peer, device_id_type=pl.DeviceIdType.MESH)

        def assign_rdma():
            return pltpu.make_async_remote_copy(
                src_ref=a_ref, dst_ref=ag,
                send_sem=send_sems.at[N_RB], recv_sem=recv_sems.at[N_RB],
                device_id=peer, device_id_type=pl.DeviceIdType.MESH)

        def out_rdma(rbv):
            rows = pl.ds(rbv * RB, RB)
            return pltpu.make_async_remote_copy(
                src_ref=obf.at[rows, :], dst_ref=orecv.at[rows, :],
                send_sem=send_sems.at[N_RB + 1 + rbv],
                recv_sem=recv_sems.at[N_RB + 1 + rbv],
                device_id=peer, device_id_type=pl.DeviceIdType.MESH)

        @pl.when(first)
        def _start_token_exchange():
            barrier = pltpu.get_barrier_semaphore()
            pl.semaphore_signal(barrier, inc=1, device_id=peer,
                                device_id_type=pl.DeviceIdType.MESH)
            pl.semaphore_wait(barrier, 1)
            xbf[:, :] = x_ref[:, :].astype(jnp.bfloat16)
            for rbv in range(N_RB):
                token_rdma(rbv).start()
            assign_rdma().start()

        e_glob = my_y * ne + e
        fcols = pl.ds(fc * F_BLK, F_BLK)

        @pl.when((slot == 0) & (rb == 0))
        def _cache_weights():
            w1bf[e, :, fcols] = w1_ref[0, :, :].astype(jnp.bfloat16)
            w2bf[e, fcols, :] = w2_ref[0, :, :].astype(jnp.bfloat16)

        w1b = w1bf[e, :, fcols]
        rows = pl.ds(rb * RB, RB)

        def moe_chunk(sl, xs, mask):
            hbuf[:, fcols] = jnp.maximum(
                jnp.dot(xs, w1b, preferred_element_type=jnp.float32), 0.0
            ).astype(jnp.bfloat16)

            @pl.when(fc == nfc - 1)
            def _second_matmul():
                o = jnp.dot(hbuf[:, :], w2bf[e, :, :],
                            preferred_element_type=jnp.float32)
                contrib = jnp.where(mask, o, 0.0)

                @pl.when(e == 0)
                def _init():
                    acc[sl, rows, :] = contrib

                @pl.when(e != 0)
                def _accum():
                    acc[sl, rows, :] = acc[sl, rows, :] + contrib

        @pl.when(slot == 0)
        def _local_block():
            moe_chunk(0, xbf[rows, :], a_ref[rows, :] == e_glob)

        @pl.when(slot == 1)
        def _remote_block():
            @pl.when((rb == 0) & (e == 0) & (fc == 0))
            def _():
                token_rdma(0).wait()
                assign_rdma().wait()

            @pl.when((rb == 1) & (e == 0) & (fc == 0))
            def _():
                token_rdma(1).wait()

            moe_chunk(1, xg[rows, :], ag[rows, :] == e_glob)

            @pl.when(last_ef)
            def _send_partial():
                obf[rows, :] = acc[1, rows, :].astype(jnp.bfloat16)
                for rbv in range(N_RB):
                    @pl.when(rb == rbv)
                    def _(rbv=rbv):
                        out_rdma(rbv).start()

        @pl.when(final)
        def _finish():
            for rbv in range(N_RB):
                out_rdma(rbv).wait()
            out_ref[:, :] = acc[0, :, :] + orecv[:, :].astype(jnp.float32)

    return pl.pallas_call(
        body,
        grid=(2, N_RB, ne, nfc),
        out_shape=jax.ShapeDtypeStruct((m, d), jnp.float32),
        in_specs=[
            pl.BlockSpec((m, d), lambda s, r, e, fc: (0, 0),
                         memory_space=pltpu.VMEM),
            pl.BlockSpec((m, 1), lambda s, r, e, fc: (0, 0),
                         memory_space=pltpu.VMEM),
            pl.BlockSpec(
                (1, d, F_BLK),
                lambda s, r, e, fc: (
                    jnp.where((s == 0) & (r == 0), e, 1),
                    0,
                    jnp.where((s == 0) & (r == 0), fc, 3),
                ),
                memory_space=pltpu.VMEM),
            pl.BlockSpec(
                (1, F_BLK, d),
                lambda s, r, e, fc: (
                    jnp.where((s == 0) & (r == 0), e, 1),
                    jnp.where((s == 0) & (r == 0), fc, 3),
                    0,
                ),
                memory_space=pltpu.VMEM),
        ],
        out_specs=pl.BlockSpec((m, d), lambda s, r, e, fc: (0, 0),
                               memory_space=pltpu.VMEM),
        scratch_shapes=[
            pltpu.VMEM((m, d), jnp.bfloat16),
            pltpu.VMEM((m, d), jnp.bfloat16),
            pltpu.VMEM((m, 1), jnp.int32),
            pltpu.VMEM((2, m, d), jnp.float32),
            pltpu.VMEM((m, d), jnp.bfloat16),
            pltpu.VMEM((m, d), jnp.bfloat16),
            pltpu.VMEM((ne, d, f), jnp.bfloat16),
            pltpu.VMEM((ne, f, d), jnp.bfloat16),
            pltpu.VMEM((RB, f), jnp.bfloat16),
            pltpu.SemaphoreType.DMA((N_RB + 1 + N_RB,)),
            pltpu.SemaphoreType.DMA((N_RB + 1 + N_RB,)),
        ],
        compiler_params=pltpu.CompilerParams(
            collective_id=0,
            dimension_semantics=("arbitrary",) * 4,
            vmem_limit_bytes=60 * 1024 * 1024,
        ),
    )(x, assign2, W1, W2)


# device time: 67815 ns/iter; 2.3760x vs baseline; 1.2661x over previous
import jax
import jax.numpy as jnp
from jax import lax
from jax.experimental import pallas as pl
from jax.experimental.pallas import tpu as pltpu

F_BLK = 1024
HR = 512
RC = 256
N_RC = 2


def kernel(x, assign, W1, W2):
    m, d = x.shape
    ne, _, f = W1.shape
    nfc = f // F_BLK
    assign2 = assign.reshape(m, 1)

    def body(x_ref, a_ref, w1_ref, w2_ref, out_ref,
             xbf, xg, abf, ag, accp, obf, oobf, xrecv, yrecv, drecv,
             w1bf, w2bf, hbuf, send_sems, recv_sems):
        s = pl.program_id(0)
        rb = pl.program_id(1)
        e = pl.program_id(2)
        fc = pl.program_id(3)
        first = (s == 0) & (rb == 0) & (e == 0) & (fc == 0)
        last_ef = (e == ne - 1) & (fc == nfc - 1)
        final = (s == 1) & (rb == N_RC - 1) & last_ef

        my_x = lax.axis_index("x")
        my_y = lax.axis_index("y")
        xpeer = (1 - my_x, my_y)
        ypeer = (my_x, 1 - my_y)
        diag = (1 - my_x, 1 - my_y)

        myrows = pl.ds(my_x * HR, HR)
        otrows = pl.ds((1 - my_x) * HR, HR)

        def token_rdma():
            return pltpu.make_async_remote_copy(
                src_ref=xbf, dst_ref=xg,
                send_sem=send_sems.at[0], recv_sem=recv_sems.at[0],
                device_id=ypeer, device_id_type=pl.DeviceIdType.MESH)

        def assign_rdma():
            return pltpu.make_async_remote_copy(
                src_ref=abf, dst_ref=ag,
                send_sem=send_sems.at[1], recv_sem=recv_sems.at[1],
                device_id=ypeer, device_id_type=pl.DeviceIdType.MESH)

        def own_rdma(rc):
            rr = pl.ds(rc * RC, RC)
            return pltpu.make_async_remote_copy(
                src_ref=oobf.at[rr, :], dst_ref=xrecv.at[rr, :],
                send_sem=send_sems.at[2 + rc], recv_sem=recv_sems.at[2 + rc],
                device_id=xpeer, device_id_type=pl.DeviceIdType.MESH)

        def peer_rdma(rc):
            rr = pl.ds(rc * RC, RC)
            return pltpu.make_async_remote_copy(
                src_ref=obf.at[rr, :], dst_ref=yrecv.at[rr, :],
                send_sem=send_sems.at[4 + rc], recv_sem=recv_sems.at[4 + rc],
                device_id=ypeer, device_id_type=pl.DeviceIdType.MESH)

        def diag_rdma(rc):
            rr = pl.ds(rc * RC, RC)
            return pltpu.make_async_remote_copy(
                src_ref=obf.at[rr, :], dst_ref=drecv.at[rr, :],
                send_sem=send_sems.at[6 + rc], recv_sem=recv_sems.at[6 + rc],
                device_id=diag, device_id_type=pl.DeviceIdType.MESH)

        @pl.when(first)
        def _start_token_exchange():
            barrier = pltpu.get_barrier_semaphore()
            for nbr in (xpeer, ypeer, diag):
                pl.semaphore_signal(barrier, inc=1, device_id=nbr,
                                    device_id_type=pl.DeviceIdType.MESH)
            pl.semaphore_wait(barrier, 3)
            xbf[:, :] = x_ref[myrows, :].astype(jnp.bfloat16)
            abf[:, :] = a_ref[myrows, :]
            token_rdma().start()
            assign_rdma().start()

        e_glob = my_y * ne + e
        fcols = pl.ds(fc * F_BLK, F_BLK)

        @pl.when((s == 0) & (rb == 0))
        def _cache_weights():
            w1bf[e, :, fcols] = w1_ref[0, :, :].astype(jnp.bfloat16)
            w2bf[e, fcols, :] = w2_ref[0, :, :].astype(jnp.bfloat16)

        w1b = w1bf[e, :, fcols]

        def moe_chunk(dst, rows, xs, mask):
            hbuf[:, fcols] = jnp.maximum(
                jnp.dot(xs, w1b, preferred_element_type=jnp.float32), 0.0
            ).astype(jnp.bfloat16)

            @pl.when(fc == nfc - 1)
            def _second_matmul():
                o = jnp.dot(hbuf[:, :], w2bf[e, :, :],
                            preferred_element_type=jnp.float32)
                contrib = jnp.where(mask, o, 0.0)

                @pl.when(e == 0)
                def _init():
                    dst[rows, :] = contrib

                @pl.when(e != 0)
                def _accum():
                    dst[rows, :] = dst[rows, :] + contrib

        crows = pl.ds(rb * RC, RC)
        gcrows = pl.ds(my_x * HR + rb * RC, RC)

        @pl.when(s == 0)
        def _own_half():
            moe_chunk(out_ref, gcrows, xbf[crows, :],
                      a_ref[gcrows, :] == e_glob)

            @pl.when(last_ef)
            def _send_own():
                oobf[crows, :] = out_ref[gcrows, :].astype(jnp.bfloat16)
                for rc in range(N_RC):
                    @pl.when(rb == rc)
                    def _(rc=rc):
                        own_rdma(rc).start()

        @pl.when(s == 1)
        def _peer_half():
            @pl.when((rb == 0) & (e == 0) & (fc == 0))
            def _():
                token_rdma().wait()
                assign_rdma().wait()

            moe_chunk(accp, crows, xg[crows, :],
                      ag[crows, :] == e_glob)

            @pl.when(last_ef)
            def _send_peer():
                obf[crows, :] = accp[crows, :].astype(jnp.bfloat16)
                for rc in range(N_RC):
                    @pl.when(rb == rc)
                    def _(rc=rc):
                        diag_rdma(rc).start()
                        peer_rdma(rc).start()

        @pl.when(final)
        def _finish():
            for rc in range(N_RC):
                own_rdma(rc).wait()
                peer_rdma(rc).wait()
                diag_rdma(rc).wait()
            out_ref[myrows, :] = (out_ref[myrows, :]
                                  + yrecv[:, :].astype(jnp.float32))
            out_ref[otrows, :] = (xrecv[:, :].astype(jnp.float32)
                                  + drecv[:, :].astype(jnp.float32))

    return pl.pallas_call(
        body,
        grid=(2, N_RC, ne, nfc),
        out_shape=jax.ShapeDtypeStruct((m, d), jnp.float32),
        in_specs=[
            pl.BlockSpec((m, d), lambda s, r, e, fc: (0, 0),
                         memory_space=pltpu.VMEM),
            pl.BlockSpec((m, 1), lambda s, r, e, fc: (0, 0),
                         memory_space=pltpu.VMEM),
            pl.BlockSpec(
                (1, d, F_BLK),
                lambda s, r, e, fc: (
                    jnp.where((s == 0) & (r == 0), e, 1),
                    0,
                    jnp.where((s == 0) & (r == 0), fc, nfc - 1),
                ),
                memory_space=pltpu.VMEM),
            pl.BlockSpec(
                (1, F_BLK, d),
                lambda s, r, e, fc: (
                    jnp.where((s == 0) & (r == 0), e, 1),
                    jnp.where((s == 0) & (r == 0), fc, nfc - 1),
                    0,
                ),
                memory_space=pltpu.VMEM),
        ],
        out_specs=pl.BlockSpec((m, d), lambda s, r, e, fc: (0, 0),
                               memory_space=pltpu.VMEM),
        scratch_shapes=[
            pltpu.VMEM((HR, d), jnp.bfloat16),
            pltpu.VMEM((HR, d), jnp.bfloat16),
            pltpu.VMEM((HR, 1), jnp.int32),
            pltpu.VMEM((HR, 1), jnp.int32),
            pltpu.VMEM((HR, d), jnp.float32),
            pltpu.VMEM((HR, d), jnp.bfloat16),
            pltpu.VMEM((HR, d), jnp.bfloat16),
            pltpu.VMEM((HR, d), jnp.bfloat16),
            pltpu.VMEM((HR, d), jnp.bfloat16),
            pltpu.VMEM((HR, d), jnp.bfloat16),
            pltpu.VMEM((ne, d, f), jnp.bfloat16),
            pltpu.VMEM((ne, f, d), jnp.bfloat16),
            pltpu.VMEM((RC, f), jnp.bfloat16),
            pltpu.SemaphoreType.DMA((8,)),
            pltpu.SemaphoreType.DMA((8,)),
        ],
        compiler_params=pltpu.CompilerParams(
            collective_id=0,
            dimension_semantics=("arbitrary",) * 4,
            vmem_limit_bytes=63 * 1024 * 1024,
        ),
    )(x, assign2, W1, W2)
